# baseline (speedup 1.0000x reference)
import jax
import jax.numpy as jnp
from jax import lax
from jax.experimental import pallas as pl
from jax.experimental.pallas import tpu as pltpu
from jax.experimental.pallas import tpu_sc as plsc

_NUM_EMB = 1000000
_D = 128
_B = 16384
_NC, _NS = 2, 16
_NW = _NC * _NS
_BPW = _B // _NW
_CH = 128
_NCH = _BPW // _CH
_NG = _CH // 16
_BITREV = [0, 8, 4, 12, 2, 10, 6, 14, 1, 9, 5, 13, 3, 11, 7, 15]


def _sc_body(idx1_hbm, idx2_hbm, table_hbm, w_hbm, out_hbm,
             idx1_v, idx2_v, w_v, rows1_v, rows2_v, out_v,
             sem1a, sem1b, sem2a, sem2b):
    wid = lax.axis_index("s") * _NC + lax.axis_index("c")
    r0 = wid * _NCH
    pltpu.sync_copy(idx1_hbm.at[pl.ds(r0, _NCH)], idx1_v)
    pltpu.sync_copy(idx2_hbm.at[pl.ds(r0, _NCH)], idx2_v)
    pltpu.sync_copy(w_hbm, w_v)

    sems1 = (sem1a, sem1b)
    sems2 = (sem2a, sem2b)

    def issue(c):
        slot = c % 2
        d1 = pltpu.async_copy(table_hbm.at[idx1_v.at[c]],
                              rows1_v.at[slot], sems1[slot])
        d2 = pltpu.async_copy(table_hbm.at[idx2_v.at[c]],
                              rows2_v.at[slot], sems2[slot])
        return d1, d2

    lane = lax.iota(jnp.int32, 16)
    zeros16 = jnp.zeros((16,), jnp.int32)
    bias = w_v[pl.ds(256, 16)].at[zeros16].get(mode="promise_in_bounds")
    maskv = {m: (lane & m) == 0 for m in (8, 4, 2, 1)}
    permv = {m: lane ^ m for m in (8, 4, 2, 1)}

    def combine(a, b, m):
        t1 = jnp.where(maskv[m], a, b)
        t2 = jnp.where(maskv[m], b, a).at[permv[m]].get(
            mode="promise_in_bounds")
        return t1 + t2

    pending = issue(0)
    for c in range(_NCH):
        slot = c % 2
        d1, d2 = pending
        if c + 1 < _NCH:
            pending = issue(c + 1)
        d1.wait()
        d2.wait()

        @plsc.parallel_loop(0, _NG)
        def group_body(g, slot=slot, c=c):
            base = g * 16
            mstage = (8, 4, 2, 1)
            stack = []

            for quarter in range(4):
                accs = [None] * 4
                for k in range(16):
                    if k < 8:
                        w = w_v[pl.ds(k * 16, 16)]
                    else:
                        w = w_v[pl.ds(128 + (k - 8) * 16, 16)]
                    for i in range(4):
                        r = base + _BITREV[quarter * 4 + i]
                        src = rows1_v if k < 8 else rows2_v
                        p = src.at[slot].at[r][pl.ds((k % 8) * 16, 16)] * w
                        accs[i] = p if accs[i] is None else accs[i] + p
                for i in range(4):
                    lvl, v = 0, accs[i]
                    while stack and stack[-1][0] == lvl:
                        _, pv = stack.pop()
                        v = combine(pv, v, mstage[lvl])
                        lvl += 1
                    stack.append((lvl, v))
            z = stack[0][1] + bias
            sig = 1.0 / (1.0 + jnp.exp(-z))
            out_v[pl.ds(c * _CH + base, 16)] = sig
    pltpu.sync_copy(out_v, out_hbm.at[pl.ds(wid * _BPW, _BPW)])


@jax.jit
def kernel(input1, input2, emb_table, fc_w, fc_b):
    idx1 = input1.astype(jnp.int32).reshape(_NW * _NCH, _CH)
    idx2 = input2.astype(jnp.int32).reshape(_NW * _NCH, _CH)
    w = jnp.concatenate(
        [fc_w.reshape(-1), fc_b.reshape(-1),
         jnp.zeros((15,), jnp.float32)]).astype(jnp.float32)
    mesh = plsc.VectorSubcoreMesh(core_axis_name="c", subcore_axis_name="s",
                                  num_cores=_NC, num_subcores=_NS)
    out = pl.kernel(
        _sc_body,
        out_type=jax.ShapeDtypeStruct((_B,), jnp.float32),
        mesh=mesh,
        compiler_params=pltpu.CompilerParams(needs_layout_passes=False),
        scratch_types=[
            pltpu.VMEM((_NCH, _CH), jnp.int32),
            pltpu.VMEM((_NCH, _CH), jnp.int32),
            pltpu.VMEM((272,), jnp.float32),
            pltpu.VMEM((2, _CH, _D), jnp.float32),
            pltpu.VMEM((2, _CH, _D), jnp.float32),
            pltpu.VMEM((_BPW,), jnp.float32),
            pltpu.SemaphoreType.DMA,
            pltpu.SemaphoreType.DMA,
            pltpu.SemaphoreType.DMA,
            pltpu.SemaphoreType.DMA,
        ],
    )(idx1, idx2, emb_table, w)
    return out.reshape(_B, 1)


# speedup vs baseline: 1.1690x; 1.1690x over previous
import jax
import jax.numpy as jnp
from jax import lax
from jax.experimental import pallas as pl
from jax.experimental.pallas import tpu as pltpu
from jax.experimental.pallas import tpu_sc as plsc

_NUM_EMB = 1000000
_D = 128
_B = 16384
_NC, _NS = 2, 16
_NW = _NC * _NS
_BPW = _B // _NW
_CH = 128
_NCH = _BPW // _CH
_NG = _CH // 16
_BITREV = [0, 8, 4, 12, 2, 10, 6, 14, 1, 9, 5, 13, 3, 11, 7, 15]


def _sc_body(idx1_hbm, idx2_hbm, table_hbm, w_hbm, out_hbm,
             idx1_v, idx2_v, w_v, rows1_v, rows2_v, out_v,
             sem1a, sem1b, sem2a, sem2b):
    wid = lax.axis_index("s") * _NC + lax.axis_index("c")
    r0 = wid * _NCH
    pltpu.sync_copy(idx1_hbm.at[pl.ds(r0, _NCH)], idx1_v)
    pltpu.sync_copy(idx2_hbm.at[pl.ds(r0, _NCH)], idx2_v)
    pltpu.sync_copy(w_hbm, w_v)

    sems1 = (sem1a, sem1b)
    sems2 = (sem2a, sem2b)

    def issue(c):
        slot = c % 2
        d1 = pltpu.async_copy(table_hbm.at[idx1_v.at[c]],
                              rows1_v.at[slot], sems1[slot])
        d2 = pltpu.async_copy(table_hbm.at[idx2_v.at[c]],
                              rows2_v.at[slot], sems2[slot])
        return d1, d2

    lane = lax.iota(jnp.int32, 16)
    zeros16 = jnp.zeros((16,), jnp.int32)
    bias = w_v[pl.ds(256, 16)].at[zeros16].get(mode="promise_in_bounds")
    maskv = {m: (lane & m) == 0 for m in (8, 4, 2, 1)}
    permv = {m: lane ^ m for m in (8, 4, 2, 1)}

    def combine(a, b, m):
        t1 = jnp.where(maskv[m], a, b)
        t2 = jnp.where(maskv[m], b, a).at[permv[m]].get(
            mode="promise_in_bounds")
        return t1 + t2

    pending = issue(0)
    for c in range(_NCH):
        slot = c % 2
        d1, d2 = pending
        if c + 1 < _NCH:
            pending = issue(c + 1)
        d1.wait()
        d2.wait()

        def group_body(g, gcarry, slot=slot, c=c):
            base = g * 16
            mstage = (8, 4, 2, 1)
            stack = []

            for quarter in range(4):
                accs = [None] * 4
                for k in range(16):
                    if k < 8:
                        w = w_v[pl.ds(k * 16, 16)]
                    else:
                        w = w_v[pl.ds(128 + (k - 8) * 16, 16)]
                    for i in range(4):
                        r = base + _BITREV[quarter * 4 + i]
                        src = rows1_v if k < 8 else rows2_v
                        p = src.at[slot].at[r][pl.ds((k % 8) * 16, 16)] * w
                        accs[i] = p if accs[i] is None else accs[i] + p
                for i in range(4):
                    lvl, v = 0, accs[i]
                    while stack and stack[-1][0] == lvl:
                        _, pv = stack.pop()
                        v = combine(pv, v, mstage[lvl])
                        lvl += 1
                    stack.append((lvl, v))
            z = stack[0][1] + bias
            sig = 1.0 / (1.0 + jnp.exp(-z))
            out_v[pl.ds(c * _CH + base, 16)] = sig
            return gcarry

        lax.fori_loop(0, _NG, group_body, 0)
    pltpu.sync_copy(out_v, out_hbm.at[pl.ds(wid * _BPW, _BPW)])


@jax.jit
def kernel(input1, input2, emb_table, fc_w, fc_b):
    idx1 = input1.astype(jnp.int32).reshape(_NW * _NCH, _CH)
    idx2 = input2.astype(jnp.int32).reshape(_NW * _NCH, _CH)
    w = jnp.concatenate(
        [fc_w.reshape(-1), fc_b.reshape(-1),
         jnp.zeros((15,), jnp.float32)]).astype(jnp.float32)
    mesh = plsc.VectorSubcoreMesh(core_axis_name="c", subcore_axis_name="s",
                                  num_cores=_NC, num_subcores=_NS)
    out = pl.kernel(
        _sc_body,
        out_type=jax.ShapeDtypeStruct((_B,), jnp.float32),
        mesh=mesh,
        compiler_params=pltpu.CompilerParams(needs_layout_passes=False),
        scratch_types=[
            pltpu.VMEM((_NCH, _CH), jnp.int32),
            pltpu.VMEM((_NCH, _CH), jnp.int32),
            pltpu.VMEM((272,), jnp.float32),
            pltpu.VMEM((2, _CH, _D), jnp.float32),
            pltpu.VMEM((2, _CH, _D), jnp.float32),
            pltpu.VMEM((_BPW,), jnp.float32),
            pltpu.SemaphoreType.DMA,
            pltpu.SemaphoreType.DMA,
            pltpu.SemaphoreType.DMA,
            pltpu.SemaphoreType.DMA,
        ],
    )(idx1, idx2, emb_table, w)
    return out.reshape(_B, 1)


# speedup vs baseline: 1.2474x; 1.0671x over previous
import jax
import jax.numpy as jnp
from jax import lax
from jax.experimental import pallas as pl
from jax.experimental.pallas import tpu as pltpu
from jax.experimental.pallas import tpu_sc as plsc

_NUM_EMB = 1000000
_D = 128
_B = 16384
_NC, _NS = 2, 16
_NW = _NC * _NS
_BPW = _B // _NW
_CH = 128
_NCH = _BPW // _CH
_NG = _CH // 16
_BITREV = [0, 8, 4, 12, 2, 10, 6, 14, 1, 9, 5, 13, 3, 11, 7, 15]


def _sc_body(idx1_hbm, idx2_hbm, table_hbm, w_hbm, out_hbm,
             idx1_v, idx2_v, w_v, rows1_v, rows2_v, out_v,
             sem1a, sem1b, sem2a, sem2b):
    wid = lax.axis_index("s") * _NC + lax.axis_index("c")
    r0 = wid * _NCH
    pltpu.sync_copy(idx1_hbm.at[pl.ds(r0, _NCH)], idx1_v)
    pltpu.sync_copy(idx2_hbm.at[pl.ds(r0, _NCH)], idx2_v)
    pltpu.sync_copy(w_hbm, w_v)

    sems1 = (sem1a, sem1b)
    sems2 = (sem2a, sem2b)

    def issue(c):
        slot = c % 2
        d1 = pltpu.async_copy(table_hbm.at[idx1_v.at[c]],
                              rows1_v.at[slot], sems1[slot])
        d2 = pltpu.async_copy(table_hbm.at[idx2_v.at[c]],
                              rows2_v.at[slot], sems2[slot])
        return d1, d2

    lane = lax.iota(jnp.int32, 16)
    zeros16 = jnp.zeros((16,), jnp.int32)
    bias = w_v[pl.ds(256, 16)].at[zeros16].get(mode="promise_in_bounds")
    maskv = {m: (lane & m) == 0 for m in (8, 4, 2, 1)}
    permv = {m: lane ^ m for m in (8, 4, 2, 1)}

    def combine(a, b, m):
        t1 = jnp.where(maskv[m], a, b)
        t2 = jnp.where(maskv[m], b, a).at[permv[m]].get(
            mode="promise_in_bounds")
        return t1 + t2

    pending = issue(0)
    for c in range(_NCH):
        slot = c % 2
        d1, d2 = pending
        if c + 1 < _NCH:
            pending = issue(c + 1)
        d1.wait()
        d2.wait()

        def group_body(g, gcarry, slot=slot, c=c):
            base = g * 16

            accs = [None] * 8
            for k in range(16):
                if k < 8:
                    w = w_v[pl.ds(k * 16, 16)]
                else:
                    w = w_v[pl.ds(128 + (k - 8) * 16, 16)]
                src = rows1_v if k < 8 else rows2_v
                off = (k % 8) * 16
                for i in range(8):
                    ra = base + _BITREV[2 * i]
                    rb = base + _BITREV[2 * i + 1]
                    pa = src.at[slot].at[ra][pl.ds(off, 16)] * w
                    pb = src.at[slot].at[rb][pl.ds(off, 16)] * w
                    l1 = combine(pa, pb, 8)
                    accs[i] = l1 if accs[i] is None else accs[i] + l1
            vs = accs
            for m in (4, 2, 1):
                vs = [combine(vs[2 * i], vs[2 * i + 1], m)
                      for i in range(len(vs) // 2)]
            z = vs[0] + bias
            sig = 1.0 / (1.0 + jnp.exp(-z))
            out_v[pl.ds(c * _CH + base, 16)] = sig
            return gcarry

        lax.fori_loop(0, _NG, group_body, 0)
    pltpu.sync_copy(out_v, out_hbm.at[pl.ds(wid * _BPW, _BPW)])


@jax.jit
def kernel(input1, input2, emb_table, fc_w, fc_b):
    idx1 = input1.astype(jnp.int32).reshape(_NW * _NCH, _CH)
    idx2 = input2.astype(jnp.int32).reshape(_NW * _NCH, _CH)
    w = jnp.concatenate(
        [fc_w.reshape(-1), fc_b.reshape(-1),
         jnp.zeros((15,), jnp.float32)]).astype(jnp.float32)
    mesh = plsc.VectorSubcoreMesh(core_axis_name="c", subcore_axis_name="s",
                                  num_cores=_NC, num_subcores=_NS)
    out = pl.kernel(
        _sc_body,
        out_type=jax.ShapeDtypeStruct((_B,), jnp.float32),
        mesh=mesh,
        compiler_params=pltpu.CompilerParams(needs_layout_passes=False),
        scratch_types=[
            pltpu.VMEM((_NCH, _CH), jnp.int32),
            pltpu.VMEM((_NCH, _CH), jnp.int32),
            pltpu.VMEM((272,), jnp.float32),
            pltpu.VMEM((2, _CH, _D), jnp.float32),
            pltpu.VMEM((2, _CH, _D), jnp.float32),
            pltpu.VMEM((_BPW,), jnp.float32),
            pltpu.SemaphoreType.DMA,
            pltpu.SemaphoreType.DMA,
            pltpu.SemaphoreType.DMA,
            pltpu.SemaphoreType.DMA,
        ],
    )(idx1, idx2, emb_table, w)
    return out.reshape(_B, 1)


# speedup vs baseline: 1.3897x; 1.1141x over previous
import jax
import jax.numpy as jnp
from jax import lax
from jax.experimental import pallas as pl
from jax.experimental.pallas import tpu as pltpu
from jax.experimental.pallas import tpu_sc as plsc

_NUM_EMB = 1000000
_D = 128
_B = 16384
_NC, _NS = 2, 16
_NW = _NC * _NS
_BPW = _B // _NW
_CH = 128
_NCH = _BPW // _CH
_NG = _CH // 16
_BITREV = [0, 8, 4, 12, 2, 10, 6, 14, 1, 9, 5, 13, 3, 11, 7, 15]


def _sc_body(idx1_hbm, idx2_hbm, table_hbm, w_hbm, out_hbm,
             idx1_v, idx2_v, w_v, rows1_v, rows2_v, out_v,
             sem1a, sem2a):
    wid = lax.axis_index("s") * _NC + lax.axis_index("c")
    r0 = wid * _NCH
    pltpu.sync_copy(idx1_hbm.at[pl.ds(r0, _NCH)], idx1_v)
    pltpu.sync_copy(idx2_hbm.at[pl.ds(r0, _NCH)], idx2_v)
    pltpu.sync_copy(w_hbm, w_v)

    def issue(c):
        p = (c & 1) * _CH
        pltpu.async_copy(table_hbm.at[idx1_v.at[c]],
                         rows1_v.at[pl.ds(p, _CH)], sem1a)
        pltpu.async_copy(table_hbm.at[idx2_v.at[c]],
                         rows2_v.at[pl.ds(p, _CH)], sem2a)

    def wait(c):
        p = (c & 1) * _CH
        pltpu.make_async_copy(table_hbm.at[idx1_v.at[c]],
                              rows1_v.at[pl.ds(p, _CH)], sem1a).wait()
        pltpu.make_async_copy(table_hbm.at[idx2_v.at[c]],
                              rows2_v.at[pl.ds(p, _CH)], sem2a).wait()

    lane = lax.iota(jnp.int32, 16)
    zeros16 = jnp.zeros((16,), jnp.int32)
    bias = w_v[pl.ds(256, 16)].at[zeros16].get(mode="promise_in_bounds")
    maskv = {m: (lane & m) == 0 for m in (8, 4, 2, 1)}
    permv = {m: lane ^ m for m in (8, 4, 2, 1)}

    def combine(a, b, m):
        t1 = jnp.where(maskv[m], a, b)
        t2 = jnp.where(maskv[m], b, a).at[permv[m]].get(
            mode="promise_in_bounds")
        return t1 + t2

    issue(0)

    def chunk_body(c, carry):
        wait(c)

        @pl.when(c < _NCH - 1)
        def _prefetch():
            issue(c + 1)

        pbase = (c & 1) * _CH

        def group_body(g, gcarry):
            base = g * 16
            rbase = pbase + base

            accs = [None] * 8
            for k in range(16):
                if k < 8:
                    w = w_v[pl.ds(k * 16, 16)]
                else:
                    w = w_v[pl.ds(128 + (k - 8) * 16, 16)]
                src = rows1_v if k < 8 else rows2_v
                off = (k % 8) * 16
                for i in range(8):
                    ra = rbase + _BITREV[2 * i]
                    rb = rbase + _BITREV[2 * i + 1]
                    pa = src.at[ra][pl.ds(off, 16)] * w
                    pb = src.at[rb][pl.ds(off, 16)] * w
                    l1 = combine(pa, pb, 8)
                    accs[i] = l1 if accs[i] is None else accs[i] + l1
            vs = accs
            for m in (4, 2, 1):
                vs = [combine(vs[2 * i], vs[2 * i + 1], m)
                      for i in range(len(vs) // 2)]
            z = vs[0] + bias
            sig = 1.0 / (1.0 + jnp.exp(-z))
            out_v[pl.ds(c * _CH + base, 16)] = sig
            return gcarry

        lax.fori_loop(0, _NG, group_body, 0)
        return carry

    lax.fori_loop(0, _NCH, chunk_body, 0)
    pltpu.sync_copy(out_v, out_hbm.at[pl.ds(wid * _BPW, _BPW)])


@jax.jit
def kernel(input1, input2, emb_table, fc_w, fc_b):
    idx1 = input1.astype(jnp.int32).reshape(_NW * _NCH, _CH)
    idx2 = input2.astype(jnp.int32).reshape(_NW * _NCH, _CH)
    w = jnp.concatenate(
        [fc_w.reshape(-1), fc_b.reshape(-1),
         jnp.zeros((15,), jnp.float32)]).astype(jnp.float32)
    mesh = plsc.VectorSubcoreMesh(core_axis_name="c", subcore_axis_name="s",
                                  num_cores=_NC, num_subcores=_NS)
    out = pl.kernel(
        _sc_body,
        out_type=jax.ShapeDtypeStruct((_B,), jnp.float32),
        mesh=mesh,
        compiler_params=pltpu.CompilerParams(needs_layout_passes=False),
        scratch_types=[
            pltpu.VMEM((_NCH, _CH), jnp.int32),
            pltpu.VMEM((_NCH, _CH), jnp.int32),
            pltpu.VMEM((272,), jnp.float32),
            pltpu.VMEM((2 * _CH, _D), jnp.float32),
            pltpu.VMEM((2 * _CH, _D), jnp.float32),
            pltpu.VMEM((_BPW,), jnp.float32),
            pltpu.SemaphoreType.DMA,
            pltpu.SemaphoreType.DMA,
        ],
    )(idx1, idx2, emb_table, w)
    return out.reshape(_B, 1)


# speedup vs baseline: 1.4604x; 1.0509x over previous
import jax
import jax.numpy as jnp
from jax import lax
from jax.experimental import pallas as pl
from jax.experimental.pallas import tpu as pltpu
from jax.experimental.pallas import tpu_sc as plsc

_NUM_EMB = 1000000
_D = 128
_B = 16384
_NC, _NS = 2, 16
_NW = _NC * _NS
_BPW = _B // _NW
_CH = 128
_NCH = _BPW // _CH
_NG = _CH // 16
_BITREV = [0, 8, 4, 12, 2, 10, 6, 14, 1, 9, 5, 13, 3, 11, 7, 15]


def _sc_body(idx1_hbm, idx2_hbm, table_hbm, w_hbm, b_hbm, out_hbm,
             idx1_v, idx2_v, w_v, b_v, rows1_v, rows2_v, out_v,
             sem1a, sem2a, semw):
    wid = lax.axis_index("s") * _NC + lax.axis_index("c")
    r0 = wid * _NCH
    di1 = pltpu.async_copy(idx1_hbm.at[pl.ds(r0, _NCH)], idx1_v, semw)
    di2 = pltpu.async_copy(idx2_hbm.at[pl.ds(r0, _NCH)], idx2_v, semw)
    dw = pltpu.async_copy(w_hbm, w_v, semw)
    db = pltpu.async_copy(b_hbm, b_v, semw)
    di1.wait()
    di2.wait()

    def issue(c):
        p = (c & 1) * _CH
        pltpu.async_copy(table_hbm.at[idx1_v.at[c]],
                         rows1_v.at[pl.ds(p, _CH)], sem1a)
        pltpu.async_copy(table_hbm.at[idx2_v.at[c]],
                         rows2_v.at[pl.ds(p, _CH)], sem2a)

    def wait(c):
        p = (c & 1) * _CH
        pltpu.make_async_copy(table_hbm.at[idx1_v.at[c]],
                              rows1_v.at[pl.ds(p, _CH)], sem1a).wait()
        pltpu.make_async_copy(table_hbm.at[idx2_v.at[c]],
                              rows2_v.at[pl.ds(p, _CH)], sem2a).wait()

    issue(0)
    dw.wait()
    db.wait()

    lane = lax.iota(jnp.int32, 16)
    zeros16 = jnp.zeros((16,), jnp.int32)
    bias = plsc.load_gather(b_v, [zeros16])
    maskv = {m: (lane & m) == 0 for m in (8, 4, 2, 1)}
    permv = {m: lane ^ m for m in (8, 4, 2, 1)}

    def combine(a, b, m):
        t1 = jnp.where(maskv[m], a, b)
        t2 = jnp.where(maskv[m], b, a).at[permv[m]].get(
            mode="promise_in_bounds")
        return t1 + t2

    def chunk_body(c, carry):
        wait(c)

        @pl.when(c < _NCH - 1)
        def _prefetch():
            issue(c + 1)

        pbase = (c & 1) * _CH

        def group_body(g, gcarry):
            base = g * 16
            rbase = pbase + base

            accs = [None] * 8
            for k in range(16):
                if k < 8:
                    w = w_v[pl.ds(k * 16, 16)]
                else:
                    w = w_v[pl.ds(128 + (k - 8) * 16, 16)]
                src = rows1_v if k < 8 else rows2_v
                off = (k % 8) * 16
                for i in range(8):
                    ra = rbase + _BITREV[2 * i]
                    rb = rbase + _BITREV[2 * i + 1]
                    pa = src.at[ra][pl.ds(off, 16)] * w
                    pb = src.at[rb][pl.ds(off, 16)] * w
                    l1 = combine(pa, pb, 8)
                    accs[i] = l1 if accs[i] is None else accs[i] + l1
            vs = accs
            for m in (4, 2, 1):
                vs = [combine(vs[2 * i], vs[2 * i + 1], m)
                      for i in range(len(vs) // 2)]
            z = vs[0] + bias
            sig = 1.0 / (1.0 + jnp.exp(-z))
            out_v[pl.ds(c * _CH + base, 16)] = sig
            return gcarry

        lax.fori_loop(0, _NG, group_body, 0)
        return carry

    lax.fori_loop(0, _NCH, chunk_body, 0)
    pltpu.sync_copy(out_v, out_hbm.at[pl.ds(wid * _BPW, _BPW)])


@jax.jit
def kernel(input1, input2, emb_table, fc_w, fc_b):
    idx1 = input1.astype(jnp.int32).reshape(_NW * _NCH, _CH)
    idx2 = input2.astype(jnp.int32).reshape(_NW * _NCH, _CH)
    w = fc_w.reshape(-1)
    mesh = plsc.VectorSubcoreMesh(core_axis_name="c", subcore_axis_name="s",
                                  num_cores=_NC, num_subcores=_NS)
    out = pl.kernel(
        _sc_body,
        out_type=jax.ShapeDtypeStruct((_B,), jnp.float32),
        mesh=mesh,
        compiler_params=pltpu.CompilerParams(needs_layout_passes=False),
        scratch_types=[
            pltpu.VMEM((_NCH, _CH), jnp.int32),
            pltpu.VMEM((_NCH, _CH), jnp.int32),
            pltpu.VMEM((256,), jnp.float32),
            pltpu.VMEM((1,), jnp.float32),
            pltpu.VMEM((2 * _CH, _D), jnp.float32),
            pltpu.VMEM((2 * _CH, _D), jnp.float32),
            pltpu.VMEM((_BPW,), jnp.float32),
            pltpu.SemaphoreType.DMA,
            pltpu.SemaphoreType.DMA,
            pltpu.SemaphoreType.DMA,
        ],
    )(idx1, idx2, emb_table, w, fc_b)
    return out.reshape(_B, 1)


# speedup vs baseline: 1.5090x; 1.0333x over previous
import jax
import jax.numpy as jnp
from jax import lax
from jax.experimental import pallas as pl
from jax.experimental.pallas import tpu as pltpu
from jax.experimental.pallas import tpu_sc as plsc

_NUM_EMB = 1000000
_D = 128
_B = 16384
_NC, _NS = 2, 16
_NW = _NC * _NS
_BPW = _B // _NW
_CH = 64
_NCH = _BPW // _CH
_NG = _CH // 16
_NSLOT = 4
_BITREV = [0, 8, 4, 12, 2, 10, 6, 14, 1, 9, 5, 13, 3, 11, 7, 15]


def _sc_body(idx1_hbm, idx2_hbm, table_hbm, w_hbm, b_hbm, out_hbm,
             idx1_v, idx2_v, w_v, b_v, rows1_v, rows2_v, out_v,
             sem1a, sem2a, semw, semo):
    wid = lax.axis_index("s") * _NC + lax.axis_index("c")
    r0 = wid * _NCH
    di1 = pltpu.async_copy(idx1_hbm.at[pl.ds(r0, _NCH)], idx1_v, semw)
    di2 = pltpu.async_copy(idx2_hbm.at[pl.ds(r0, _NCH)], idx2_v, semw)
    dw = pltpu.async_copy(w_hbm, w_v, semw)
    db = pltpu.async_copy(b_hbm, b_v, semw)
    di1.wait()
    di2.wait()

    def issue(c):
        p = (c & (_NSLOT - 1)) * _CH
        pltpu.async_copy(table_hbm.at[idx1_v.at[c]],
                         rows1_v.at[pl.ds(p, _CH)], sem1a)
        pltpu.async_copy(table_hbm.at[idx2_v.at[c]],
                         rows2_v.at[pl.ds(p, _CH)], sem2a)

    def wait(c):
        p = (c & (_NSLOT - 1)) * _CH
        pltpu.make_async_copy(table_hbm.at[idx1_v.at[c]],
                              rows1_v.at[pl.ds(p, _CH)], sem1a).wait()
        pltpu.make_async_copy(table_hbm.at[idx2_v.at[c]],
                              rows2_v.at[pl.ds(p, _CH)], sem2a).wait()

    issue(0)
    issue(1)
    dw.wait()
    db.wait()

    lane = lax.iota(jnp.int32, 16)
    zeros16 = jnp.zeros((16,), jnp.int32)
    bias = plsc.load_gather(b_v, [zeros16])
    maskv = {m: (lane & m) == 0 for m in (8, 4, 2, 1)}
    permv = {m: lane ^ m for m in (8, 4, 2, 1)}

    def combine(a, b, m):
        t1 = jnp.where(maskv[m], a, b)
        t2 = jnp.where(maskv[m], b, a).at[permv[m]].get(
            mode="promise_in_bounds")
        return t1 + t2

    def chunk_body(c, carry):
        wait(c)

        @pl.when(c < _NCH - 2)
        def _prefetch():
            issue(c + 2)

        pbase = (c & (_NSLOT - 1)) * _CH

        def group_body(g, gcarry):
            base = g * 16
            rbase = pbase + base

            accs = [None] * 8
            for k in range(16):
                if k < 8:
                    w = w_v[pl.ds(k * 16, 16)]
                else:
                    w = w_v[pl.ds(128 + (k - 8) * 16, 16)]
                src = rows1_v if k < 8 else rows2_v
                off = (k % 8) * 16
                for i in range(8):
                    ra = rbase + _BITREV[2 * i]
                    rb = rbase + _BITREV[2 * i + 1]
                    pa = src.at[ra][pl.ds(off, 16)] * w
                    pb = src.at[rb][pl.ds(off, 16)] * w
                    l1 = combine(pa, pb, 8)
                    accs[i] = l1 if accs[i] is None else accs[i] + l1
            vs = accs
            for m in (4, 2, 1):
                vs = [combine(vs[2 * i], vs[2 * i + 1], m)
                      for i in range(len(vs) // 2)]
            z = vs[0] + bias
            sig = 1.0 / (1.0 + jnp.exp(-z))
            out_v[pl.ds(c * _CH + base, 16)] = sig
            return gcarry

        lax.fori_loop(0, _NG, group_body, 0)
        pltpu.async_copy(out_v.at[pl.ds(c * _CH, _CH)],
                         out_hbm.at[pl.ds(wid * _BPW + c * _CH, _CH)], semo)
        return carry

    lax.fori_loop(0, _NCH, chunk_body, 0)
    pltpu.make_async_copy(out_v, out_hbm.at[pl.ds(wid * _BPW, _BPW)],
                          semo).wait()


@jax.jit
def kernel(input1, input2, emb_table, fc_w, fc_b):
    idx1 = input1.astype(jnp.int32).reshape(_NW * _NCH, _CH)
    idx2 = input2.astype(jnp.int32).reshape(_NW * _NCH, _CH)
    w = fc_w.reshape(-1)
    mesh = plsc.VectorSubcoreMesh(core_axis_name="c", subcore_axis_name="s",
                                  num_cores=_NC, num_subcores=_NS)
    out = pl.kernel(
        _sc_body,
        out_type=jax.ShapeDtypeStruct((_B,), jnp.float32),
        mesh=mesh,
        compiler_params=pltpu.CompilerParams(needs_layout_passes=False),
        scratch_types=[
            pltpu.VMEM((_NCH, _CH), jnp.int32),
            pltpu.VMEM((_NCH, _CH), jnp.int32),
            pltpu.VMEM((256,), jnp.float32),
            pltpu.VMEM((1,), jnp.float32),
            pltpu.VMEM((_NSLOT * _CH, _D), jnp.float32),
            pltpu.VMEM((_NSLOT * _CH, _D), jnp.float32),
            pltpu.VMEM((_BPW,), jnp.float32),
            pltpu.SemaphoreType.DMA,
            pltpu.SemaphoreType.DMA,
            pltpu.SemaphoreType.DMA,
            pltpu.SemaphoreType.DMA,
        ],
    )(idx1, idx2, emb_table, w, fc_b)
    return out.reshape(_B, 1)
